# baseline (speedup 1.0000x reference)
import jax
import jax.numpy as jnp
import numpy as np
from jax import lax
from jax.experimental import pallas as pl
from jax.experimental.pallas import tpu as pltpu

_GAMMA = 0.5
_EPS = 1e-06
_BT = 1024


def _fused_body(bij_ref, t_ref, arow_ref, a_ref, k_ref, kt_ref, out_ref,
                acc_ref, deg_ref, m_ref, degt_ref):
    l = pl.program_id(0)
    b = bij_ref[0, l]
    i = bij_ref[1, l]
    j = bij_ref[2, l]
    nb = acc_ref.shape[0]
    ni = deg_ref.shape[0] // nb

    @pl.when(l < nb)
    def _init():
        acc_ref[pl.ds(b, 1), :] = jnp.zeros((1, acc_ref.shape[1]), jnp.float32)
        m_ref[b] = 0.0
        degt_ref[b] = 1.0
        for q in range(ni):
            deg_ref[pl.ds(b * ni + q, 1), :] = jnp.zeros(
                (1, deg_ref.shape[1]), jnp.float32)

    a = a_ref[0]
    ones = jnp.ones((1, _BT), jnp.float32)
    cs = lax.dot_general(ones, a, (((1,), (0,)), ((), ())),
                         preferred_element_type=jnp.float32)
    deg_ref[pl.ds(b * ni + j, 1), :] += cs

    @pl.when(i != j)
    def _rows():
        rs = lax.dot_general(ones, a, (((1,), (1,)), ((), ())),
                             preferred_element_type=jnp.float32)
        deg_ref[pl.ds(b * ni + i, 1), :] += rs

    @pl.when(j == ni - 1)
    def _phase_b():
        deg_i = deg_ref[pl.ds(b * ni + i, 1), :] + _EPS
        ris = lax.rsqrt(deg_i)
        u = arow_ref[0] * ris
        k = k_ref[0]
        acc_ref[pl.ds(b, 1), :] += lax.dot_general(
            u, k, (((1,), (0,)), ((), ())),
            preferred_element_type=jnp.float32)
        m_ref[b] = jnp.maximum(m_ref[b], jnp.max(jnp.abs(u)))

        t = t_ref[0]
        start = i * _BT
        in_chunk = jnp.logical_and(t >= start, t < start + _BT)

        @pl.when(in_chunk)
        def _corr():
            loc = t - start
            lane = lax.broadcasted_iota(jnp.int32, (1, _BT), 1)
            sel = lane == loc
            u_t = jnp.sum(jnp.where(sel, u, 0.0))
            krow = k_ref[0, pl.ds(loc, 1), :]
            acc_ref[pl.ds(b, 1), :] += u_t * (kt_ref[0] - krow)
            degt_ref[b] = jnp.sum(jnp.where(sel, deg_i, 0.0))

    rd = lax.rsqrt(degt_ref[b])
    mask = jnp.where(m_ref[b] * rd > 1e-9, 1.0, 0.0)
    out_ref[0] = acc_ref[pl.ds(b, 1), :] * (rd * jnp.sqrt(_GAMMA) * mask)


def kernel(A_rel, K_past, k_t, t):
    B, T, D = K_past.shape
    ni = T // _BT
    pairs = [(i, j) for i in range(ni) for j in range(ni) if j >= i]
    bij = [(b, i, j) for (i, j) in pairs for b in range(B)]
    bij = jnp.asarray(np.array(bij, dtype=np.int32).T)
    nt = bij.shape[1]
    t_i = jnp.asarray(t, jnp.int32).reshape(1)
    arow3 = lax.dynamic_slice(A_rel, (0, t_i[0], 0), (B, 1, T))

    grid_spec = pltpu.PrefetchScalarGridSpec(
        num_scalar_prefetch=2,
        grid=(nt,),
        in_specs=[
            pl.BlockSpec((1, 1, _BT),
                         lambda l, bij, t: (bij[0, l], 0, bij[1, l])),
            pl.BlockSpec((1, _BT, _BT),
                         lambda l, bij, t: (bij[0, l], bij[1, l], bij[2, l])),
            pl.BlockSpec((1, _BT, D),
                         lambda l, bij, t: (bij[0, l], bij[1, l], 0)),
            pl.BlockSpec((1, 1, D), lambda l, bij, t: (bij[0, l], 0, 0)),
        ],
        out_specs=pl.BlockSpec((1, 1, D), lambda l, bij, t: (bij[0, l], 0, 0)),
        scratch_shapes=[
            pltpu.VMEM((B, D), jnp.float32),
            pltpu.VMEM((B * ni, _BT), jnp.float32),
            pltpu.SMEM((B,), jnp.float32),
            pltpu.SMEM((B,), jnp.float32),
        ],
    )
    out = pl.pallas_call(
        _fused_body,
        grid_spec=grid_spec,
        out_shape=jax.ShapeDtypeStruct((B, 1, D), jnp.float32),
    )(bij, t_i, arow3, A_rel, K_past, k_t.reshape(B, 1, D))
    return out.reshape(B, D)


# speedup vs baseline: 1.3092x; 1.3092x over previous
import jax
import jax.numpy as jnp
import numpy as np
from jax import lax
from jax.experimental import pallas as pl
from jax.experimental.pallas import tpu as pltpu

_GAMMA = 0.5
_EPS = 1e-06
_BT = 1024


def _fused_body(ij_ref, t_ref, arow_ref, a_ref, k_ref, kt_ref, out_ref,
                acc_ref, deg_ref, m_ref, degt_ref):
    l = pl.program_id(1)
    nl = pl.num_programs(1)
    i = ij_ref[0, l]
    j = ij_ref[1, l]
    ni = deg_ref.shape[0]

    @pl.when(l == 0)
    def _init():
        acc_ref[...] = jnp.zeros_like(acc_ref)
        deg_ref[...] = jnp.zeros_like(deg_ref)
        m_ref[0] = 0.0
        degt_ref[0] = 1.0

    a = a_ref[0]
    ones = jnp.ones((1, _BT), jnp.float32)
    cs = lax.dot_general(ones, a, (((1,), (0,)), ((), ())),
                         preferred_element_type=jnp.float32)
    deg_ref[pl.ds(j, 1), :] += cs

    @pl.when(i != j)
    def _rows():
        rs = lax.dot_general(ones, a, (((1,), (1,)), ((), ())),
                             preferred_element_type=jnp.float32)
        deg_ref[pl.ds(i, 1), :] += rs

    @pl.when(j == ni - 1)
    def _phase_b():
        deg_i = deg_ref[pl.ds(i, 1), :] + _EPS
        ris = lax.rsqrt(deg_i)
        u = arow_ref[0] * ris
        k = k_ref[0]
        acc_ref[...] += lax.dot_general(
            u, k, (((1,), (0,)), ((), ())),
            preferred_element_type=jnp.float32)
        m_ref[0] = jnp.maximum(m_ref[0], jnp.max(jnp.abs(u)))

        t = t_ref[0]
        start = i * _BT
        in_chunk = jnp.logical_and(t >= start, t < start + _BT)

        @pl.when(in_chunk)
        def _corr():
            loc = t - start
            lane = lax.broadcasted_iota(jnp.int32, (1, _BT), 1)
            sel = lane == loc
            u_t = jnp.sum(jnp.where(sel, u, 0.0))
            krow = k_ref[0, pl.ds(loc, 1), :]
            acc_ref[...] += u_t * (kt_ref[0] - krow)
            degt_ref[0] = jnp.sum(jnp.where(sel, deg_i, 0.0))

    @pl.when(l == nl - 1)
    def _fin():
        rd = lax.rsqrt(degt_ref[0])
        mask = jnp.where(m_ref[0] * rd > 1e-9, 1.0, 0.0)
        out_ref[0] = acc_ref[...] * (rd * jnp.sqrt(_GAMMA) * mask)


def kernel(A_rel, K_past, k_t, t):
    B, T, D = K_past.shape
    ni = T // _BT
    pairs = [(i, j) for i in range(ni) for j in range(ni) if j >= i]
    ij = jnp.asarray(np.array(pairs, dtype=np.int32).T)
    nt = len(pairs)
    t_i = jnp.asarray(t, jnp.int32).reshape(1)
    arow3 = lax.dynamic_slice(A_rel, (0, t_i[0], 0), (B, 1, T))

    grid_spec = pltpu.PrefetchScalarGridSpec(
        num_scalar_prefetch=2,
        grid=(B, nt),
        in_specs=[
            pl.BlockSpec((1, 1, _BT), lambda b, l, ij, t: (b, 0, ij[0, l])),
            pl.BlockSpec((1, _BT, _BT), lambda b, l, ij, t: (b, ij[0, l], ij[1, l])),
            pl.BlockSpec((1, _BT, D), lambda b, l, ij, t: (b, ij[0, l], 0)),
            pl.BlockSpec((1, 1, D), lambda b, l, ij, t: (b, 0, 0)),
        ],
        out_specs=pl.BlockSpec((1, 1, D), lambda b, l, ij, t: (b, 0, 0)),
        scratch_shapes=[
            pltpu.VMEM((1, D), jnp.float32),
            pltpu.VMEM((ni, T // ni), jnp.float32),
            pltpu.SMEM((1,), jnp.float32),
            pltpu.SMEM((1,), jnp.float32),
        ],
    )
    out = pl.pallas_call(
        _fused_body,
        grid_spec=grid_spec,
        out_shape=jax.ShapeDtypeStruct((B, 1, D), jnp.float32),
    )(ij, t_i, arow3, A_rel, K_past, k_t.reshape(B, 1, D))
    return out.reshape(B, D)
